# baseline (speedup 1.0000x reference)
import functools
from math import sqrt

import jax
import jax.numpy as jnp
from jax import lax
from jax.experimental import pallas as pl
from jax.experimental.pallas import tpu as pltpu
from jax.experimental.pallas import tpu_sc as plsc

L = 16
NC, NS = 2, 16
NW = NC * NS
CH = 16
NBUF = 5
LOOKAHEAD = 3
BT = 2
G = 8


def _build_sc(B, S, V, D):
    P = S // NW
    NCH = P // CH
    NV = D // L
    T = NCH * B
    scale = float(sqrt(D))
    mesh = plsc.VectorSubcoreMesh(core_axis_name="c", subcore_axis_name="s")

    @functools.partial(
        pl.kernel,
        out_type=jax.ShapeDtypeStruct((B * S, D), jnp.float32),
        mesh=mesh,
        scratch_types=[
            pltpu.VMEM((B, P), jnp.int32),
            [pltpu.VMEM((CH, D), jnp.float32) for _ in range(2)],
            [pltpu.VMEM((CH, D), jnp.float32) for _ in range(NBUF)],
            pltpu.SemaphoreType.DMA,
            pltpu.SemaphoreType.DMA,
            pltpu.SemaphoreType.DMA,
        ],
    )
    def k(xf_hbm, table_hbm, pe_hbm, out_hbm, idx_v, pe_v, gbuf, gsem, wsem, psem):
        wid = lax.axis_index("s") * NC + lax.axis_index("c")
        pos_base = wid * P

        for b in range(B):
            pltpu.sync_copy(xf_hbm.at[pl.ds(b * S + pos_base, P)], idx_v.at[b])

        def row_base(t):
            c, b = divmod(t, B)
            return b * S + pos_base + c * CH

        def start_gather(t):
            c, b = divmod(t, B)
            return pltpu.async_copy(
                table_hbm.at[idx_v.at[b, pl.ds(c * CH, CH)]],
                gbuf[t % NBUF], gsem)

        def start_pe(c):
            return pltpu.async_copy(
                pe_hbm.at[pl.ds(pos_base + c * CH, CH)], pe_v[c % 2], psem)

        pe_copies = [start_pe(0)]
        gathers = [start_gather(t) for t in range(LOOKAHEAD)]
        writes = []

        for t in range(T):
            c, b = divmod(t, B)
            if b == 0:
                pe_copies[c].wait()
                if c + 1 < NCH:
                    pe_copies.append(start_pe(c + 1))
            gathers[t].wait()
            g = gbuf[t % NBUF]
            p = pe_v[c % 2]

            def body(j, _, g=g, p=p):
                sl = pl.ds(j * L, L)
                for r in range(CH):
                    g[r, sl] = g[r, sl] * scale + p[r, sl]
                return _

            lax.fori_loop(0, NV, body, 0)
            writes.append(pltpu.async_copy(
                g, out_hbm.at[pl.ds(row_base(t), CH)], wsem))
            if t + LOOKAHEAD < T:
                prev = t + LOOKAHEAD - NBUF
                if prev >= 0:
                    writes[prev].wait()
                gathers.append(start_gather(t + LOOKAHEAD))
        for t in range(max(0, T - NBUF), T):
            writes[t].wait()

    return k


def _build_tc(NR, S, V, D):
    scale = float(sqrt(D))
    SL, LN = 8, D // 8

    def body(idx_ref, *refs):
        rows = refs[:G]
        pe_ref = refs[G]
        out_ref = refs[G + 1]
        gathered = jnp.concatenate([r[...] for r in rows], axis=0)
        out_ref[...] = gathered * scale + pe_ref[...]

    table_specs = [
        pl.BlockSpec((1, SL, LN), (lambda i, idx_ref, g=g: (idx_ref[i * G + g], 0, 0)))
        for g in range(G)
    ]
    pe_spec = pl.BlockSpec((G, SL, LN), lambda i, idx_ref: (i % (S // G), 0, 0))
    out_spec = pl.BlockSpec((G, SL, LN), lambda i, idx_ref: (i, 0, 0))
    return pl.pallas_call(
        body,
        grid_spec=pltpu.PrefetchScalarGridSpec(
            num_scalar_prefetch=1,
            grid=(NR // G,),
            in_specs=table_specs + [pe_spec],
            out_specs=out_spec,
        ),
        out_shape=jax.ShapeDtypeStruct((NR, SL, LN), jnp.float32),
    )


def kernel(x, embed_table, pe):
    B, S = x.shape
    V, D = embed_table.shape
    xf = x.reshape(B * S).astype(jnp.int32)
    pe2 = pe.reshape(pe.shape[-2], pe.shape[-1])[:S]

    parts = []
    if BT > 0:
        tc = _build_tc(BT * S, S, V, D)
        table3 = embed_table.reshape(V, 8, D // 8)
        pe3 = pe2.reshape(S, 8, D // 8)
        parts.append(
            tc(xf[: BT * S], *([table3] * G), pe3).reshape(BT * S, D))
    if BT < B:
        sc = _build_sc(B - BT, S, V, D)
        parts.append(sc(xf[BT * S:], embed_table, pe2))
    out = parts[0] if len(parts) == 1 else jnp.concatenate(parts, axis=0)
    return out.reshape(B, S, D)


# speedup vs baseline: 4.6653x; 4.6653x over previous
import functools
from math import sqrt

import jax
import jax.numpy as jnp
from jax import lax
from jax.experimental import pallas as pl
from jax.experimental.pallas import tpu as pltpu
from jax.experimental.pallas import tpu_sc as plsc

L = 16
NC, NS = 2, 16
NW = NC * NS
CH = 16
NBUF = 5
LOOKAHEAD = 3
BT = 2
G = 8


def _build_sc(B, S, V, D):
    P = S // NW
    NCH = P // CH
    NV = D // L
    T = NCH * B
    scale = float(sqrt(D))
    mesh = plsc.VectorSubcoreMesh(core_axis_name="c", subcore_axis_name="s")

    @functools.partial(
        pl.kernel,
        out_type=jax.ShapeDtypeStruct((B * S, D), jnp.float32),
        mesh=mesh,
        scratch_types=[
            pltpu.VMEM((B, P), jnp.int32),
            [pltpu.VMEM((CH, D), jnp.float32) for _ in range(2)],
            [pltpu.VMEM((CH, D), jnp.float32) for _ in range(NBUF)],
            pltpu.SemaphoreType.DMA,
            pltpu.SemaphoreType.DMA,
            pltpu.SemaphoreType.DMA,
        ],
    )
    def k(xf_hbm, table_hbm, pe_hbm, out_hbm, idx_v, pe_v, gbuf, gsem, wsem, psem):
        wid = lax.axis_index("s") * NC + lax.axis_index("c")
        pos_base = wid * P

        for b in range(B):
            pltpu.sync_copy(xf_hbm.at[pl.ds(b * S + pos_base, P)], idx_v.at[b])

        def row_base(t):
            c, b = divmod(t, B)
            return b * S + pos_base + c * CH

        def start_gather(t):
            c, b = divmod(t, B)
            return pltpu.async_copy(
                table_hbm.at[idx_v.at[b, pl.ds(c * CH, CH)]],
                gbuf[t % NBUF], gsem)

        def start_pe(c):
            return pltpu.async_copy(
                pe_hbm.at[pl.ds(pos_base + c * CH, CH)], pe_v[c % 2], psem)

        pe_copies = [start_pe(0)]
        gathers = [start_gather(t) for t in range(LOOKAHEAD)]
        writes = []

        for t in range(T):
            c, b = divmod(t, B)
            if b == 0:
                pe_copies[c].wait()
                if c + 1 < NCH:
                    pe_copies.append(start_pe(c + 1))
            gathers[t].wait()
            g = gbuf[t % NBUF]
            p = pe_v[c % 2]

            def body(j, _, g=g, p=p):
                sl = pl.ds(j * L, L)
                for r in range(CH):
                    g[r, sl] = g[r, sl] * scale + p[r, sl]
                return _

            lax.fori_loop(0, NV, body, 0)
            writes.append(pltpu.async_copy(
                g, out_hbm.at[pl.ds(row_base(t), CH)], wsem))
            if t + LOOKAHEAD < T:
                prev = t + LOOKAHEAD - NBUF
                if prev >= 0:
                    writes[prev].wait()
                gathers.append(start_gather(t + LOOKAHEAD))
        for t in range(max(0, T - NBUF), T):
            writes[t].wait()

    return k


def _build_tc(NR, S, V, D):
    scale = float(sqrt(D))
    CHT = 16
    NB2 = 4
    LA = 3
    NCHT = NR // CHT

    def body(idx_ref, table, pe_ref, out, gbuf, obuf, gsem, wsem):
        t = pl.program_id(0)

        def issue(c, slot):
            for j in range(CHT):
                pltpu.make_async_copy(
                    table.at[pl.ds(idx_ref[c * CHT + j], 1)],
                    gbuf.at[slot, pl.ds(j, 1)], gsem.at[slot]).start()

        @pl.when(t == 0)
        def _prime():
            for c in range(LA):
                issue(c, c)

        slot = lax.rem(t, NB2)
        pltpu.make_async_copy(
            table.at[pl.ds(0, CHT)], gbuf.at[slot], gsem.at[slot]).wait()

        oslot = lax.rem(t, 2)

        @pl.when(t >= 2)
        def _drain_write():
            pltpu.make_async_copy(
                obuf.at[0], out.at[pl.ds(0, CHT)], wsem.at[oslot]).wait()

        pos = lax.rem(t * CHT, S)
        obuf[oslot] = gbuf[slot] * scale + pe_ref[pl.ds(pos, CHT), :]
        pltpu.make_async_copy(
            obuf.at[oslot], out.at[pl.ds(t * CHT, CHT)], wsem.at[oslot]).start()

        @pl.when(t + LA < NCHT)
        def _next():
            issue(t + LA, lax.rem(t + LA, NB2))

        @pl.when(t == NCHT - 1)
        def _final_drain():
            for o in range(2):
                pltpu.make_async_copy(
                    obuf.at[0], out.at[pl.ds(0, CHT)], wsem.at[o]).wait()

    return pl.pallas_call(
        body,
        grid_spec=pltpu.PrefetchScalarGridSpec(
            num_scalar_prefetch=1,
            grid=(NCHT,),
            in_specs=[
                pl.BlockSpec(memory_space=pltpu.MemorySpace.HBM),
                pl.BlockSpec((S, D), lambda i, idx_ref: (0, 0)),
            ],
            out_specs=pl.BlockSpec(memory_space=pltpu.MemorySpace.HBM),
            scratch_shapes=[
                pltpu.VMEM((NB2, CHT, D), jnp.float32),
                pltpu.VMEM((2, CHT, D), jnp.float32),
                pltpu.SemaphoreType.DMA((NB2,)),
                pltpu.SemaphoreType.DMA((2,)),
            ],
        ),
        out_shape=jax.ShapeDtypeStruct((NR, D), jnp.float32),
        compiler_params=pltpu.CompilerParams(
            dimension_semantics=("arbitrary",)),
    )


def kernel(x, embed_table, pe):
    B, S = x.shape
    V, D = embed_table.shape
    xf = x.reshape(B * S).astype(jnp.int32)
    pe2 = pe.reshape(pe.shape[-2], pe.shape[-1])[:S]

    parts = []
    if BT > 0:
        tc = _build_tc(BT * S, S, V, D)
        parts.append(tc(xf[: BT * S], embed_table, pe2))
    if BT < B:
        sc = _build_sc(B - BT, S, V, D)
        parts.append(sc(xf[BT * S:], embed_table, pe2))
    out = parts[0] if len(parts) == 1 else jnp.concatenate(parts, axis=0)
    return out.reshape(B, S, D)
